# Initial kernel scaffold; baseline (speedup 1.0000x reference)
import jax
import jax.numpy as jnp
from jax.experimental import pallas as pl


def kernel(vertex_feat, neighbors_idx, valid_lens, W, B_w):
    raise NotImplementedError("write your pallas kernel here")


# speedup vs baseline: 6.1349x; 6.1349x over previous
import functools

import jax
import jax.numpy as jnp
from jax import lax
from jax.experimental import pallas as pl
from jax.experimental.pallas import tpu as pltpu
from jax.experimental.pallas import tpu_sc as plsc

_N = 10000
_K = 32
_D = 128
_H = 128
_NW = 32
_ROWS_PER_W = 320
_N_PAD = _NW * _ROWS_PER_W
_G = 4
_CHUNKS = _ROWS_PER_W // _G
_VPR = _D // 16


def _sc_gather_sum(vf, idx3):
    mesh = plsc.VectorSubcoreMesh(core_axis_name="c", subcore_axis_name="s")

    @functools.partial(
        pl.kernel,
        out_type=jax.ShapeDtypeStruct((_N_PAD, _D), jnp.float32),
        mesh=mesh,
        scratch_types=[
            pltpu.VMEM((_CHUNKS, _G * _K), jnp.int32),
            pltpu.VMEM((_G * _K, _D), jnp.float32),
            pltpu.VMEM((_ROWS_PER_W, _D), jnp.float32),
            pltpu.SemaphoreType.DMA,
        ],
    )
    def gather_sum(vf_hbm, idx_hbm, out_hbm, idx_v, rows_v, out_v, sem):
        wid = lax.axis_index("s") * 2 + lax.axis_index("c")
        pltpu.sync_copy(idx_hbm.at[wid], idx_v)

        def chunk_body(g, carry):
            pltpu.async_copy(vf_hbm.at[idx_v.at[g]], rows_v, sem).wait()
            for n in range(_G):
                base = n * _K

                def row_body(r, acc):
                    return tuple(
                        acc[c] + rows_v[base + r, pl.ds(c * 16, 16)]
                        for c in range(_VPR)
                    )

                acc = lax.fori_loop(
                    1, _K, row_body,
                    tuple(rows_v[base, pl.ds(c * 16, 16)] for c in range(_VPR)),
                )
                row = g * _G + n
                for c in range(_VPR):
                    out_v[row, pl.ds(c * 16, 16)] = acc[c]
            return carry

        lax.fori_loop(0, _CHUNKS, chunk_body, 0)
        pltpu.sync_copy(out_v, out_hbm.at[pl.ds(wid * _ROWS_PER_W, _ROWS_PER_W)])

    return gather_sum(vf, idx3)


def _tc_update(agg, vf, vl, W, B_w):
    R = 1000

    def body(agg_ref, vf_ref, vl_ref, w_ref, b_ref, out_ref):
        vlf = vl_ref[...].astype(jnp.float32)
        vlf = jnp.where(vlf == 0.0, 1.0, vlf)
        x = agg_ref[...] / vlf
        y = jnp.dot(x, w_ref[...], preferred_element_type=jnp.float32)
        y = y + jnp.dot(vf_ref[...], b_ref[...], preferred_element_type=jnp.float32)
        out_ref[...] = jnp.maximum(y, 0.0)

    return pl.pallas_call(
        body,
        grid=(_N // R,),
        in_specs=[
            pl.BlockSpec((R, _D), lambda i: (i, 0)),
            pl.BlockSpec((R, _D), lambda i: (i, 0)),
            pl.BlockSpec((R, 1), lambda i: (i, 0)),
            pl.BlockSpec((_D, _H), lambda i: (0, 0)),
            pl.BlockSpec((_D, _H), lambda i: (0, 0)),
        ],
        out_specs=pl.BlockSpec((R, _H), lambda i: (i, 0)),
        out_shape=jax.ShapeDtypeStruct((_N, _H), jnp.float32),
    )(agg, vf, vl, W, B_w)


def kernel(vertex_feat, neighbors_idx, valid_lens, W, B_w):
    vf = vertex_feat[0]
    idx = neighbors_idx[0].reshape(-1)
    idx = jnp.concatenate(
        [idx, jnp.zeros(((_N_PAD - _N) * _K,), jnp.int32)])
    idx3 = idx.reshape(_NW, _CHUNKS, _G * _K)
    agg = _sc_gather_sum(vf, idx3)
    out = _tc_update(agg[:_N], vf, valid_lens[0][:, None], W, B_w)
    return out[None]


# speedup vs baseline: 7.1409x; 1.1640x over previous
import functools

import jax
import jax.numpy as jnp
from jax import lax
from jax.experimental import pallas as pl
from jax.experimental.pallas import tpu as pltpu
from jax.experimental.pallas import tpu_sc as plsc

_N = 10000
_K = 32
_D = 128
_H = 128
_NW = 32
_ROWS_PER_W = 320
_N_PAD = _NW * _ROWS_PER_W
_G = 4
_CHUNKS = _ROWS_PER_W // _G
_VPR = _D // 16


def _sc_gather_sum(vf, idx3):
    mesh = plsc.VectorSubcoreMesh(core_axis_name="c", subcore_axis_name="s")

    @functools.partial(
        pl.kernel,
        out_type=jax.ShapeDtypeStruct((_N_PAD, _D), jnp.float32),
        mesh=mesh,
        scratch_types=[
            pltpu.VMEM((_CHUNKS, _G * _K), jnp.int32),
            pltpu.VMEM((_G * _K, _D), jnp.float32),
            pltpu.VMEM((_G * _K, _D), jnp.float32),
            pltpu.VMEM((_ROWS_PER_W, _D), jnp.float32),
            pltpu.SemaphoreType.DMA,
            pltpu.SemaphoreType.DMA,
        ],
    )
    def gather_sum(vf_hbm, idx_hbm, out_hbm, idx_v, rows_a, rows_b, out_v,
                   sem_a, sem_b):
        wid = lax.axis_index("s") * 2 + lax.axis_index("c")
        pltpu.sync_copy(idx_hbm.at[wid], idx_v)

        def start(g, rows, sem):
            pltpu.async_copy(vf_hbm.at[idx_v.at[g]], rows, sem)

        def wait(rows, sem):
            pltpu.make_async_copy(vf_hbm.at[idx_v.at[0]], rows, sem).wait()

        def accum(rows, out_base):
            for n in range(_G):
                base = n * _K

                def row_body(r, acc):
                    return tuple(
                        acc[c] + rows[base + r, pl.ds(c * 16, 16)]
                        for c in range(_VPR)
                    )

                acc = lax.fori_loop(
                    1, _K, row_body,
                    tuple(rows[base, pl.ds(c * 16, 16)] for c in range(_VPR)),
                )
                row = out_base + n
                for c in range(_VPR):
                    out_v[row, pl.ds(c * 16, 16)] = acc[c]

        pairs = _CHUNKS // 2
        start(0, rows_a, sem_a)

        def pair_body(t, carry):
            g0 = 2 * t
            start(g0 + 1, rows_b, sem_b)
            wait(rows_a, sem_a)
            accum(rows_a, g0 * _G)

            @pl.when(t < pairs - 1)
            def _():
                start(g0 + 2, rows_a, sem_a)

            wait(rows_b, sem_b)
            accum(rows_b, (g0 + 1) * _G)
            return carry

        lax.fori_loop(0, pairs, pair_body, 0)
        pltpu.sync_copy(out_v, out_hbm.at[pl.ds(wid * _ROWS_PER_W, _ROWS_PER_W)])

    return gather_sum(vf, idx3)


def _tc_update(agg, vf, vl, W, B_w):
    R = 1000

    def body(agg_ref, vf_ref, vl_ref, w_ref, b_ref, out_ref):
        vlf = vl_ref[...].astype(jnp.float32)
        vlf = jnp.where(vlf == 0.0, 1.0, vlf)
        x = agg_ref[...] / vlf
        y = jnp.dot(x, w_ref[...], preferred_element_type=jnp.float32)
        y = y + jnp.dot(vf_ref[...], b_ref[...], preferred_element_type=jnp.float32)
        out_ref[...] = jnp.maximum(y, 0.0)

    return pl.pallas_call(
        body,
        grid=(_N // R,),
        in_specs=[
            pl.BlockSpec((R, _D), lambda i: (i, 0)),
            pl.BlockSpec((R, _D), lambda i: (i, 0)),
            pl.BlockSpec((R, 1), lambda i: (i, 0)),
            pl.BlockSpec((_D, _H), lambda i: (0, 0)),
            pl.BlockSpec((_D, _H), lambda i: (0, 0)),
        ],
        out_specs=pl.BlockSpec((R, _H), lambda i: (i, 0)),
        out_shape=jax.ShapeDtypeStruct((_N, _H), jnp.float32),
    )(agg, vf, vl, W, B_w)


def kernel(vertex_feat, neighbors_idx, valid_lens, W, B_w):
    vf = vertex_feat[0]
    idx = neighbors_idx[0].reshape(-1)
    idx = jnp.concatenate(
        [idx, jnp.zeros(((_N_PAD - _N) * _K,), jnp.int32)])
    idx3 = idx.reshape(_NW, _CHUNKS, _G * _K)
    agg = _sc_gather_sum(vf, idx3)
    out = _tc_update(agg[:_N], vf, valid_lens[0][:, None], W, B_w)
    return out[None]
